# Initial kernel scaffold; baseline (speedup 1.0000x reference)
import jax
import jax.numpy as jnp
from jax.experimental import pallas as pl


def kernel(x, edge_index, W1, b1, W2, b2, Wl, bl):
    raise NotImplementedError("write your pallas kernel here")


# speedup vs baseline: 8.0116x; 8.0116x over previous
import functools

import jax
import jax.numpy as jnp
from jax import lax
from jax.experimental import pallas as pl
from jax.experimental.pallas import tpu as pltpu
from jax.experimental.pallas import tpu_sc as plsc

_N = 10000
_E = 320000
_D = 128
_H = 128
_C = 64

_NC = 2
_NS = 16
_NW = _NC * _NS

_NP = 10240
_RPW = _NP // _NS
_CH = 64
_NCH = 160
_EP = _NW * _NCH * _CH

_mesh = plsc.VectorSubcoreMesh(core_axis_name="c", subcore_axis_name="s")


@functools.partial(
    pl.kernel,
    out_type=jax.ShapeDtypeStruct((_NC, _NP), jnp.float32),
    mesh=_mesh,
    scratch_types=[
        pltpu.VMEM((2, _CH), jnp.int32),
        pltpu.VMEM((_CH,), jnp.float32),
        pltpu.VMEM((_RPW,), jnp.float32),
        pltpu.VMEM_SHARED((_NP,), jnp.float32),
    ],
)
def _deg_kernel(edges_hbm, deg_out, ec, ones_v, zeros_v, acc_sh):
    c = lax.axis_index("c")
    s = lax.axis_index("s")
    wid = c * _NS + s

    one = jnp.full((16,), 1.0, jnp.float32)
    z = jnp.zeros((16,), jnp.float32)
    for i in range(_CH // 16):
        ones_v[pl.ds(i * 16, 16)] = one
    for i in range(_RPW // 16):
        zeros_v[pl.ds(i * 16, 16)] = z

    pltpu.sync_copy(zeros_v, acc_sh.at[pl.ds(s * _RPW, _RPW)])
    plsc.subcore_barrier()

    def body(j, carry):
        pltpu.sync_copy(edges_hbm.at[wid, j], ec)
        pltpu.sync_copy(ones_v, acc_sh.at[ec.at[1]], add=True)
        return carry

    lax.fori_loop(0, _NCH, body, 0)
    plsc.subcore_barrier()
    pltpu.sync_copy(acc_sh.at[pl.ds(s * _RPW, _RPW)],
                    deg_out.at[c, pl.ds(s * _RPW, _RPW)])


@functools.partial(
    pl.kernel,
    out_type=jax.ShapeDtypeStruct((_NC, _NP, _H), jnp.float32),
    mesh=_mesh,
    scratch_types=[
        pltpu.VMEM((2, _CH), jnp.int32),
        pltpu.VMEM((2, _CH), jnp.int32),
        pltpu.VMEM((_CH, _H), jnp.float32),
        pltpu.VMEM((_CH, _H), jnp.float32),
        pltpu.SemaphoreType.DMA,
        pltpu.SemaphoreType.DMA,
        pltpu.VMEM_SHARED((_NP, _H), jnp.float32),
    ],
)
def _agg_kernel(h_hbm, edges_hbm, out_hbm,
                ec0, ec1, rows0, rows1, sem0, sem1, acc_sh):
    c = lax.axis_index("c")
    s = lax.axis_index("s")
    wid = c * _NS + s

    z = jnp.zeros((16,), jnp.float32)
    for i in range(_CH):
        for k in range(_H // 16):
            rows0[i, pl.ds(k * 16, 16)] = z

    for b in range(_RPW // _CH):
        pltpu.sync_copy(rows0,
                        acc_sh.at[pl.ds(s * _RPW + b * _CH, _CH)])
    plsc.subcore_barrier()

    def body(j, carry):
        pltpu.sync_copy(edges_hbm.at[wid, 2 * j], ec0)
        cp0 = pltpu.async_copy(h_hbm.at[ec0.at[0]], rows0, sem0)
        pltpu.sync_copy(edges_hbm.at[wid, 2 * j + 1], ec1)
        cp1 = pltpu.async_copy(h_hbm.at[ec1.at[0]], rows1, sem1)
        cp0.wait()
        pltpu.sync_copy(rows0, acc_sh.at[ec0.at[1]], add=True)
        cp1.wait()
        pltpu.sync_copy(rows1, acc_sh.at[ec1.at[1]], add=True)
        return carry

    lax.fori_loop(0, _NCH // 2, body, 0)
    plsc.subcore_barrier()
    pltpu.sync_copy(acc_sh.at[pl.ds(s * _RPW, _RPW)],
                    out_hbm.at[c, pl.ds(s * _RPW, _RPW)])


_R = 512
_G = _NP // _R


def _k1_body(deg_ref, x_ref, w1_ref, hs_ref, dinv_ref):
    deg = deg_ref[0, :] + deg_ref[1, :] + 1.0
    dinv = lax.rsqrt(deg)
    h = jnp.dot(x_ref[...], w1_ref[...], preferred_element_type=jnp.float32)
    hs_ref[...] = h * dinv[:, None]
    dinv_ref[...] = dinv


def _tc_layer1(deg_parts, x_p, w1):
    return pl.pallas_call(
        _k1_body,
        grid=(_G,),
        in_specs=[
            pl.BlockSpec((_NC, _R), lambda i: (0, i)),
            pl.BlockSpec((_R, _D), lambda i: (i, 0)),
            pl.BlockSpec((_D, _H), lambda i: (0, 0)),
        ],
        out_specs=[
            pl.BlockSpec((_R, _H), lambda i: (i, 0)),
            pl.BlockSpec((_R,), lambda i: (i,)),
        ],
        out_shape=[
            jax.ShapeDtypeStruct((_NP, _H), jnp.float32),
            jax.ShapeDtypeStruct((_NP,), jnp.float32),
        ],
    )(deg_parts, x_p, w1)


def _k2_body(agg_ref, hs_ref, dinv_ref, b1_ref, w2_ref, out_ref):
    dinv = dinv_ref[...]
    z = (agg_ref[0] + agg_ref[1] + hs_ref[...]) * dinv[:, None] + b1_ref[...]
    t = jnp.tanh(z)
    h2 = jnp.dot(t, w2_ref[...], preferred_element_type=jnp.float32)
    out_ref[...] = h2 * dinv[:, None]


def _tc_layer2(agg, hs, dinv, b1, w2):
    return pl.pallas_call(
        _k2_body,
        grid=(_G,),
        in_specs=[
            pl.BlockSpec((_NC, _R, _H), lambda i: (0, i, 0)),
            pl.BlockSpec((_R, _H), lambda i: (i, 0)),
            pl.BlockSpec((_R,), lambda i: (i,)),
            pl.BlockSpec((1, _H), lambda i: (0, 0)),
            pl.BlockSpec((_H, _H), lambda i: (0, 0)),
        ],
        out_specs=pl.BlockSpec((_R, _H), lambda i: (i, 0)),
        out_shape=jax.ShapeDtypeStruct((_NP, _H), jnp.float32),
    )(agg, hs, dinv, b1.reshape(1, _H), w2)


def _k3_body(agg_ref, hs_ref, dinv_ref, b2_ref, wl_ref, bl_ref, out_ref):
    dinv = dinv_ref[...]
    z = (agg_ref[0] + agg_ref[1] + hs_ref[...]) * dinv[:, None] + b2_ref[...]
    t = jnp.tanh(z)
    out_ref[...] = (jnp.dot(t, wl_ref[...], preferred_element_type=jnp.float32)
                    + bl_ref[...])


def _tc_head(agg, hs, dinv, b2, wl, bl):
    return pl.pallas_call(
        _k3_body,
        grid=(_G,),
        in_specs=[
            pl.BlockSpec((_NC, _R, _H), lambda i: (0, i, 0)),
            pl.BlockSpec((_R, _H), lambda i: (i, 0)),
            pl.BlockSpec((_R,), lambda i: (i,)),
            pl.BlockSpec((1, _H), lambda i: (0, 0)),
            pl.BlockSpec((_H, _C), lambda i: (0, 0)),
            pl.BlockSpec((1, _C), lambda i: (0, 0)),
        ],
        out_specs=pl.BlockSpec((_R, _C), lambda i: (i, 0)),
        out_shape=jax.ShapeDtypeStruct((_NP, _C), jnp.float32),
    )(agg, hs, dinv, b2.reshape(1, _H), wl, bl.reshape(1, _C))


def kernel(x, edge_index, W1, b1, W2, b2, Wl, bl):
    src = edge_index[0]
    dst = edge_index[1]
    npad = _EP - _E
    pad_src = jnp.full((npad,), _N, jnp.int32)
    pad_dst = _N + (jnp.arange(npad, dtype=jnp.int32) % (_NP - _N))
    src_p = jnp.concatenate([src, pad_src]).reshape(_NW, _NCH, 1, _CH)
    dst_p = jnp.concatenate([dst, pad_dst]).reshape(_NW, _NCH, 1, _CH)
    edges_p = jnp.concatenate([src_p, dst_p], axis=2)
    x_p = jnp.pad(x, ((0, _NP - _N), (0, 0)))

    deg_parts = _deg_kernel(edges_p)
    hs1, dinv = _tc_layer1(deg_parts, x_p, W1)
    agg1 = _agg_kernel(hs1, edges_p)
    hs2 = _tc_layer2(agg1, hs1, dinv, b1, W2)
    agg2 = _agg_kernel(hs2, edges_p)
    out = _tc_head(agg2, hs2, dinv, b2, Wl, bl)
    return out[:_N]
